# baseline (speedup 1.0000x reference)
import functools

import jax
import jax.numpy as jnp
from jax import lax
from jax.experimental import pallas as pl
from jax.experimental.pallas import tpu as pltpu
from jax.experimental.pallas import tpu_sc as plsc

NUM_EMB = 1024
DIM = 64
COMMIT = 0.25
TILE_N = 512
N_TOTAL = 18432

_NC = 2
_NS = 16
_NW = _NC * _NS
_B_PER_W = N_TOTAL // _NW


def _vq_body(x_ref, cb_ref, idx_ref, loss_ref, perp_ref,
             counts_ref, lsum_ref):
    step = pl.program_id(0)
    nsteps = pl.num_programs(0)
    x = x_ref[...]
    cb = cb_ref[...]
    cb2h = 0.5 * jnp.sum(cb * cb, axis=1)
    xc = lax.dot_general(x, cb, (((1,), (1,)), ((), ())),
                         preferred_element_type=jnp.float32)
    s = xc - cb2h[None, :]
    smax = jnp.max(s, axis=1, keepdims=True)
    kiota = lax.broadcasted_iota(jnp.int32, s.shape, 1)
    idx = jnp.min(jnp.where(s == smax, kiota, NUM_EMB), axis=1)
    idx_ref[0, 0, :] = idx
    onehot = (kiota == idx[:, None]).astype(jnp.float32)
    part_loss = jnp.sum(x * x) - 2.0 * jnp.sum(smax)
    part_counts = jnp.sum(onehot, axis=0)[None, :]

    @pl.when(step == 0)
    def _():
        counts_ref[...] = part_counts
        lsum_ref[0] = part_loss

    @pl.when(step != 0)
    def _():
        counts_ref[...] += part_counts
        lsum_ref[0] += part_loss

    @pl.when(step == nsteps - 1)
    def _():
        n_total = nsteps * TILE_N
        p = counts_ref[...] * (1.0 / n_total)
        perp_ref[0, 0] = jnp.exp(-jnp.sum(p * jnp.log(p + 1e-10)))
        loss_ref[0, 0] = (1.0 + COMMIT) * lsum_ref[0] / (n_total * DIM)


_SC_MESH = plsc.VectorSubcoreMesh(core_axis_name="c", subcore_axis_name="s")


@functools.partial(
    pl.kernel,
    mesh=_SC_MESH,
    out_type=jax.ShapeDtypeStruct((N_TOTAL, 2 * DIM), jnp.float32),
    scratch_types=[
        pltpu.VMEM((_B_PER_W,), jnp.int32),
        pltpu.VMEM((_B_PER_W, 2 * DIM), jnp.float32),
        pltpu.SemaphoreType.DMA,
    ],
)
def _sc_gather(idx_hbm, table_hbm, out_hbm, idx_v, rows_v, sem):
    wid = lax.axis_index("s") * _NC + lax.axis_index("c")
    base = wid * _B_PER_W
    pltpu.sync_copy(idx_hbm.at[pl.ds(base, _B_PER_W)], idx_v)
    pltpu.async_copy(table_hbm.at[idx_v], rows_v, sem).wait()
    pltpu.sync_copy(rows_v, out_hbm.at[pl.ds(base, _B_PER_W)])


def kernel(inputs, codebook):
    flat = inputs.reshape(-1, DIM)
    n = flat.shape[0]
    grid = (n // TILE_N,)
    idx3, loss, perp = pl.pallas_call(
        _vq_body,
        grid=grid,
        in_specs=[
            pl.BlockSpec((TILE_N, DIM), lambda i: (i, 0)),
            pl.BlockSpec((NUM_EMB, DIM), lambda i: (0, 0)),
        ],
        out_specs=[
            pl.BlockSpec((1, 1, TILE_N), lambda i: (i, 0, 0)),
            pl.BlockSpec(memory_space=pltpu.SMEM),
            pl.BlockSpec(memory_space=pltpu.SMEM),
        ],
        out_shape=[
            jax.ShapeDtypeStruct((n // TILE_N, 1, TILE_N), jnp.int32),
            jax.ShapeDtypeStruct((1, 1), jnp.float32),
            jax.ShapeDtypeStruct((1, 1), jnp.float32),
        ],
        scratch_shapes=[
            pltpu.VMEM((1, NUM_EMB), jnp.float32),
            pltpu.SMEM((1,), jnp.float32),
        ],
        compiler_params=pltpu.CompilerParams(
            dimension_semantics=("arbitrary",)),
    )(flat, codebook)
    idx = idx3.reshape(-1)
    cb_pad = jnp.pad(codebook, ((0, 0), (0, DIM)))
    q = _sc_gather(idx, cb_pad)[:, :DIM]
    return (q.reshape(inputs.shape), loss[0, 0], perp[0, 0], idx)


# speedup vs baseline: 1.3251x; 1.3251x over previous
import jax
import jax.numpy as jnp
from jax import lax
from jax.experimental import pallas as pl
from jax.experimental.pallas import tpu as pltpu

NUM_EMB = 1024
DIM = 64
COMMIT = 0.25
TILE_N = 512


def _vq_body(x_ref, cb_ref, q_ref, idx_ref, loss_ref, perp_ref,
             cb2h_ref, fiota_ref, counts_ref, lsum_ref):
    step = pl.program_id(0)
    nsteps = pl.num_programs(0)
    x = x_ref[...]
    cb = cb_ref[...]

    @pl.when(step == 0)
    def _():
        cb2h_ref[...] = 0.5 * jnp.sum(cb * cb, axis=1)[None, :]
        fiota_ref[...] = lax.broadcasted_iota(
            jnp.int32, (1, NUM_EMB), 1).astype(jnp.float32)

    xc = lax.dot_general(x, cb, (((1,), (1,)), ((), ())),
                         preferred_element_type=jnp.float32)
    s = xc - cb2h_ref[...]
    smax = jnp.max(s, axis=1, keepdims=True)
    fiota = fiota_ref[...]
    idxf = jnp.min(jnp.where(s == smax, fiota, 131072.0), axis=1)
    idx_ref[0, 0, :] = idxf.astype(jnp.int32)
    onehot = (fiota == idxf[:, None]).astype(jnp.float32)
    q_ref[...] = lax.dot_general(onehot, cb, (((1,), (0,)), ((), ())),
                                 preferred_element_type=jnp.float32)
    part_loss = jnp.sum(x * x) - 2.0 * jnp.sum(smax)
    part_counts = jnp.sum(onehot, axis=0)[None, :]

    @pl.when(step == 0)
    def _():
        counts_ref[...] = part_counts
        lsum_ref[0] = part_loss

    @pl.when(step != 0)
    def _():
        counts_ref[...] += part_counts
        lsum_ref[0] += part_loss

    @pl.when(step == nsteps - 1)
    def _():
        n_total = nsteps * TILE_N
        p = counts_ref[...] * (1.0 / n_total)
        perp_ref[0, 0] = jnp.exp(-jnp.sum(p * jnp.log(p + 1e-10)))
        loss_ref[0, 0] = (1.0 + COMMIT) * lsum_ref[0] / (n_total * DIM)


def kernel(inputs, codebook):
    flat = inputs.reshape(-1, DIM)
    n = flat.shape[0]
    grid = (n // TILE_N,)
    q, idx3, loss, perp = pl.pallas_call(
        _vq_body,
        grid=grid,
        in_specs=[
            pl.BlockSpec((TILE_N, DIM), lambda i: (i, 0)),
            pl.BlockSpec((NUM_EMB, DIM), lambda i: (0, 0)),
        ],
        out_specs=[
            pl.BlockSpec((TILE_N, DIM), lambda i: (i, 0)),
            pl.BlockSpec((1, 1, TILE_N), lambda i: (i, 0, 0)),
            pl.BlockSpec(memory_space=pltpu.SMEM),
            pl.BlockSpec(memory_space=pltpu.SMEM),
        ],
        out_shape=[
            jax.ShapeDtypeStruct((n, DIM), jnp.float32),
            jax.ShapeDtypeStruct((n // TILE_N, 1, TILE_N), jnp.int32),
            jax.ShapeDtypeStruct((1, 1), jnp.float32),
            jax.ShapeDtypeStruct((1, 1), jnp.float32),
        ],
        scratch_shapes=[
            pltpu.VMEM((1, NUM_EMB), jnp.float32),
            pltpu.VMEM((1, NUM_EMB), jnp.float32),
            pltpu.VMEM((1, NUM_EMB), jnp.float32),
            pltpu.SMEM((1,), jnp.float32),
        ],
        compiler_params=pltpu.CompilerParams(
            dimension_semantics=("arbitrary",)),
    )(flat, codebook)
    return (q.reshape(inputs.shape), loss[0, 0], perp[0, 0],
            idx3.reshape(-1))


# speedup vs baseline: 1.3616x; 1.0275x over previous
import jax
import jax.numpy as jnp
from jax import lax
from jax.experimental import pallas as pl
from jax.experimental.pallas import tpu as pltpu

NUM_EMB = 1024
DIM = 64
COMMIT = 0.25
TILE_N = 512


def _vq_body(x_ref, cb_ref, q_ref, idx_ref, loss_ref, perp_ref,
             cb2h_ref, fiota_ref, counts_ref, lsum_ref):
    step = pl.program_id(0)
    nsteps = pl.num_programs(0)
    x = x_ref[...]
    cb = cb_ref[...]

    @pl.when(step == 0)
    def _():
        cb2h_ref[...] = 0.5 * jnp.sum(cb * cb, axis=1)[None, :]
        fiota_ref[...] = lax.broadcasted_iota(
            jnp.int32, (1, NUM_EMB), 1).astype(jnp.float32)

    xc = lax.dot_general(x, cb, (((1,), (1,)), ((), ())),
                         preferred_element_type=jnp.float32)
    s = xc - cb2h_ref[...]
    smax = jnp.max(s, axis=1, keepdims=True)
    fiota = fiota_ref[...]
    idxf = jnp.min(jnp.where(s == smax, fiota, 131072.0), axis=1)
    onehot = (fiota == idxf[:, None]).astype(jnp.float32)
    idx_row = lax.dot_general(fiota, onehot, (((1,), (1,)), ((), ())),
                              preferred_element_type=jnp.float32)
    idx_ref[0, :, :] = idx_row.astype(jnp.int32)
    q_ref[...] = lax.dot_general(onehot, cb, (((1,), (0,)), ((), ())),
                                 preferred_element_type=jnp.float32)
    part_loss = jnp.sum(x * x) - 2.0 * jnp.sum(smax)
    ones_row = jnp.ones((1, TILE_N), jnp.float32)
    part_counts = lax.dot_general(ones_row, onehot, (((1,), (0,)), ((), ())),
                                  preferred_element_type=jnp.float32)

    @pl.when(step == 0)
    def _():
        counts_ref[...] = part_counts
        lsum_ref[0] = part_loss

    @pl.when(step != 0)
    def _():
        counts_ref[...] += part_counts
        lsum_ref[0] += part_loss

    @pl.when(step == nsteps - 1)
    def _():
        n_total = nsteps * TILE_N
        p = counts_ref[...] * (1.0 / n_total)
        perp_ref[0, 0] = jnp.exp(-jnp.sum(p * jnp.log(p + 1e-10)))
        loss_ref[0, 0] = (1.0 + COMMIT) * lsum_ref[0] / (n_total * DIM)


def kernel(inputs, codebook):
    flat = inputs.reshape(-1, DIM)
    n = flat.shape[0]
    grid = (n // TILE_N,)
    q, idx3, loss, perp = pl.pallas_call(
        _vq_body,
        grid=grid,
        in_specs=[
            pl.BlockSpec((TILE_N, DIM), lambda i: (i, 0)),
            pl.BlockSpec((NUM_EMB, DIM), lambda i: (0, 0)),
        ],
        out_specs=[
            pl.BlockSpec((TILE_N, DIM), lambda i: (i, 0)),
            pl.BlockSpec((1, 1, TILE_N), lambda i: (i, 0, 0)),
            pl.BlockSpec(memory_space=pltpu.SMEM),
            pl.BlockSpec(memory_space=pltpu.SMEM),
        ],
        out_shape=[
            jax.ShapeDtypeStruct((n, DIM), jnp.float32),
            jax.ShapeDtypeStruct((n // TILE_N, 1, TILE_N), jnp.int32),
            jax.ShapeDtypeStruct((1, 1), jnp.float32),
            jax.ShapeDtypeStruct((1, 1), jnp.float32),
        ],
        scratch_shapes=[
            pltpu.VMEM((1, NUM_EMB), jnp.float32),
            pltpu.VMEM((1, NUM_EMB), jnp.float32),
            pltpu.VMEM((1, NUM_EMB), jnp.float32),
            pltpu.SMEM((1,), jnp.float32),
        ],
        compiler_params=pltpu.CompilerParams(
            dimension_semantics=("arbitrary",)),
    )(flat, codebook)
    return (q.reshape(inputs.shape), loss[0, 0], perp[0, 0],
            idx3.reshape(-1))


# speedup vs baseline: 1.5774x; 1.1585x over previous
import jax
import jax.numpy as jnp
from jax import lax
from jax.experimental import pallas as pl
from jax.experimental.pallas import tpu as pltpu

NUM_EMB = 1024
DIM = 64
COMMIT = 0.25
TILE_N = 2304


def _vq_body(x_ref, cb_ref, q_ref, idx_ref, loss_ref, perp_ref,
             cb2h_ref, fiota_ref, counts_ref, lsum_ref):
    step = pl.program_id(0)
    nsteps = pl.num_programs(0)
    x = x_ref[...]
    cb = cb_ref[...]

    @pl.when(step == 0)
    def _():
        cb2h_ref[...] = 0.5 * jnp.sum(cb * cb, axis=1)[None, :]
        fiota_ref[...] = lax.broadcasted_iota(
            jnp.int32, (1, NUM_EMB), 1).astype(jnp.float32)

    xc = lax.dot_general(x, cb, (((1,), (1,)), ((), ())),
                         preferred_element_type=jnp.float32)
    s = xc - cb2h_ref[...]
    smax = jnp.max(s, axis=1, keepdims=True)
    fiota = fiota_ref[...]
    idxf = jnp.min(jnp.where(s == smax, fiota, 131072.0), axis=1)
    onehot = (fiota == idxf[:, None]).astype(jnp.float32)
    idx_row = lax.dot_general(fiota, onehot, (((1,), (1,)), ((), ())),
                              preferred_element_type=jnp.float32)
    idx_ref[0, :, :] = idx_row.astype(jnp.int32)
    q_ref[...] = lax.dot_general(onehot, cb, (((1,), (0,)), ((), ())),
                                 preferred_element_type=jnp.float32)
    part_loss = jnp.sum(x * x) - 2.0 * jnp.sum(smax)
    ones_row = jnp.ones((1, TILE_N), jnp.float32)
    part_counts = lax.dot_general(ones_row, onehot, (((1,), (0,)), ((), ())),
                                  preferred_element_type=jnp.float32)

    @pl.when(step == 0)
    def _():
        counts_ref[...] = part_counts
        lsum_ref[0] = part_loss

    @pl.when(step != 0)
    def _():
        counts_ref[...] += part_counts
        lsum_ref[0] += part_loss

    @pl.when(step == nsteps - 1)
    def _():
        n_total = nsteps * TILE_N
        p = counts_ref[...] * (1.0 / n_total)
        perp_ref[0, 0] = jnp.exp(-jnp.sum(p * jnp.log(p + 1e-10)))
        loss_ref[0, 0] = (1.0 + COMMIT) * lsum_ref[0] / (n_total * DIM)


def kernel(inputs, codebook):
    flat = inputs.reshape(-1, DIM)
    n = flat.shape[0]
    grid = (n // TILE_N,)
    q, idx3, loss, perp = pl.pallas_call(
        _vq_body,
        grid=grid,
        in_specs=[
            pl.BlockSpec((TILE_N, DIM), lambda i: (i, 0)),
            pl.BlockSpec((NUM_EMB, DIM), lambda i: (0, 0)),
        ],
        out_specs=[
            pl.BlockSpec((TILE_N, DIM), lambda i: (i, 0)),
            pl.BlockSpec((1, 1, TILE_N), lambda i: (i, 0, 0)),
            pl.BlockSpec(memory_space=pltpu.SMEM),
            pl.BlockSpec(memory_space=pltpu.SMEM),
        ],
        out_shape=[
            jax.ShapeDtypeStruct((n, DIM), jnp.float32),
            jax.ShapeDtypeStruct((n // TILE_N, 1, TILE_N), jnp.int32),
            jax.ShapeDtypeStruct((1, 1), jnp.float32),
            jax.ShapeDtypeStruct((1, 1), jnp.float32),
        ],
        scratch_shapes=[
            pltpu.VMEM((1, NUM_EMB), jnp.float32),
            pltpu.VMEM((1, NUM_EMB), jnp.float32),
            pltpu.VMEM((1, NUM_EMB), jnp.float32),
            pltpu.SMEM((1,), jnp.float32),
        ],
        compiler_params=pltpu.CompilerParams(
            dimension_semantics=("arbitrary",)),
    )(flat, codebook)
    return (q.reshape(inputs.shape), loss[0, 0], perp[0, 0],
            idx3.reshape(-1))


# speedup vs baseline: 1.5779x; 1.0004x over previous
import jax
import jax.numpy as jnp
from jax import lax
from jax.experimental import pallas as pl
from jax.experimental.pallas import tpu as pltpu

NUM_EMB = 1024
DIM = 64
COMMIT = 0.25
TILE_N = 3072


def _vq_body(x_ref, cb_ref, q_ref, idx_ref, loss_ref, perp_ref,
             cb2h_ref, fiota_ref, counts_ref, lsum_ref):
    step = pl.program_id(0)
    nsteps = pl.num_programs(0)
    x = x_ref[...]
    cb = cb_ref[...]

    @pl.when(step == 0)
    def _():
        cb2h_ref[...] = 0.5 * jnp.sum(cb * cb, axis=1)[None, :]
        fiota_ref[...] = lax.broadcasted_iota(
            jnp.int32, (1, NUM_EMB), 1).astype(jnp.float32)

    xc = lax.dot_general(x, cb, (((1,), (1,)), ((), ())),
                         preferred_element_type=jnp.float32)
    s = xc - cb2h_ref[...]
    smax = jnp.max(s, axis=1, keepdims=True)
    fiota = fiota_ref[...]
    idxf = jnp.min(jnp.where(s == smax, fiota, 131072.0), axis=1)
    onehot = (fiota == idxf[:, None]).astype(jnp.float32)
    idx_row = lax.dot_general(fiota, onehot, (((1,), (1,)), ((), ())),
                              preferred_element_type=jnp.float32)
    idx_ref[0, :, :] = idx_row.astype(jnp.int32)
    q_ref[...] = lax.dot_general(onehot, cb, (((1,), (0,)), ((), ())),
                                 preferred_element_type=jnp.float32)
    part_loss = jnp.sum(x * x) - 2.0 * jnp.sum(smax)
    ones_row = jnp.ones((1, TILE_N), jnp.float32)
    part_counts = lax.dot_general(ones_row, onehot, (((1,), (0,)), ((), ())),
                                  preferred_element_type=jnp.float32)

    @pl.when(step == 0)
    def _():
        counts_ref[...] = part_counts
        lsum_ref[0] = part_loss

    @pl.when(step != 0)
    def _():
        counts_ref[...] += part_counts
        lsum_ref[0] += part_loss

    @pl.when(step == nsteps - 1)
    def _():
        n_total = nsteps * TILE_N
        p = counts_ref[...] * (1.0 / n_total)
        perp_ref[0, 0] = jnp.exp(-jnp.sum(p * jnp.log(p + 1e-10)))
        loss_ref[0, 0] = (1.0 + COMMIT) * lsum_ref[0] / (n_total * DIM)


def kernel(inputs, codebook):
    flat = inputs.reshape(-1, DIM)
    n = flat.shape[0]
    grid = (n // TILE_N,)
    q, idx3, loss, perp = pl.pallas_call(
        _vq_body,
        grid=grid,
        in_specs=[
            pl.BlockSpec((TILE_N, DIM), lambda i: (i, 0)),
            pl.BlockSpec((NUM_EMB, DIM), lambda i: (0, 0)),
        ],
        out_specs=[
            pl.BlockSpec((TILE_N, DIM), lambda i: (i, 0)),
            pl.BlockSpec((1, 1, TILE_N), lambda i: (i, 0, 0)),
            pl.BlockSpec(memory_space=pltpu.SMEM),
            pl.BlockSpec(memory_space=pltpu.SMEM),
        ],
        out_shape=[
            jax.ShapeDtypeStruct((n, DIM), jnp.float32),
            jax.ShapeDtypeStruct((n // TILE_N, 1, TILE_N), jnp.int32),
            jax.ShapeDtypeStruct((1, 1), jnp.float32),
            jax.ShapeDtypeStruct((1, 1), jnp.float32),
        ],
        scratch_shapes=[
            pltpu.VMEM((1, NUM_EMB), jnp.float32),
            pltpu.VMEM((1, NUM_EMB), jnp.float32),
            pltpu.VMEM((1, NUM_EMB), jnp.float32),
            pltpu.SMEM((1,), jnp.float32),
        ],
        compiler_params=pltpu.CompilerParams(
            dimension_semantics=("arbitrary",)),
    )(flat, codebook)
    return (q.reshape(inputs.shape), loss[0, 0], perp[0, 0],
            idx3.reshape(-1))


# speedup vs baseline: 1.6481x; 1.0444x over previous
import jax
import jax.numpy as jnp
from jax import lax
from jax.experimental import pallas as pl
from jax.experimental.pallas import tpu as pltpu

NUM_EMB = 1024
DIM = 64
COMMIT = 0.25
TILE_N = 2304


def _vq_body(x_ref, cb_ref, q_ref, idx_ref, loss_ref, perp_ref,
             cba_ref, fiota_ref, counts_ref, lsum_ref):
    step = pl.program_id(0)
    nsteps = pl.num_programs(0)
    x = x_ref[...]
    cb = cb_ref[...]

    @pl.when(step == 0)
    def _():
        cba_ref[:, :DIM] = cb
        cba_ref[:, DIM:] = -0.5 * jnp.sum(cb * cb, axis=1, keepdims=True)
        fiota_ref[...] = lax.broadcasted_iota(
            jnp.int32, (1, NUM_EMB), 1).astype(jnp.float32)

    x_aug = jnp.concatenate([x, jnp.ones((TILE_N, 1), jnp.float32)], axis=1)
    s = lax.dot_general(x_aug, cba_ref[...], (((1,), (1,)), ((), ())),
                        preferred_element_type=jnp.float32)
    smax = jnp.max(s, axis=1, keepdims=True)
    fiota = fiota_ref[...]
    idxf = jnp.min(jnp.where(s == smax, fiota, 131072.0), axis=1)
    onehot = (fiota == idxf[:, None]).astype(jnp.float32)
    idx_row = lax.dot_general(fiota, onehot, (((1,), (1,)), ((), ())),
                              preferred_element_type=jnp.float32)
    idx_ref[0, :, :] = idx_row.astype(jnp.int32)
    q_ref[...] = lax.dot_general(onehot, cb, (((1,), (0,)), ((), ())),
                                 preferred_element_type=jnp.float32)
    part_loss = jnp.sum(x * x) - 2.0 * jnp.sum(smax)
    ones_row = jnp.ones((1, TILE_N), jnp.float32)
    part_counts = lax.dot_general(ones_row, onehot, (((1,), (0,)), ((), ())),
                                  preferred_element_type=jnp.float32)

    @pl.when(step == 0)
    def _():
        counts_ref[...] = part_counts
        lsum_ref[0] = part_loss

    @pl.when(step != 0)
    def _():
        counts_ref[...] += part_counts
        lsum_ref[0] += part_loss

    @pl.when(step == nsteps - 1)
    def _():
        n_total = nsteps * TILE_N
        p = counts_ref[...] * (1.0 / n_total)
        perp_ref[0, 0] = jnp.exp(-jnp.sum(p * jnp.log(p + 1e-10)))
        loss_ref[0, 0] = (1.0 + COMMIT) * lsum_ref[0] / (n_total * DIM)


def kernel(inputs, codebook):
    flat = inputs.reshape(-1, DIM)
    n = flat.shape[0]
    grid = (n // TILE_N,)
    q, idx3, loss, perp = pl.pallas_call(
        _vq_body,
        grid=grid,
        in_specs=[
            pl.BlockSpec((TILE_N, DIM), lambda i: (i, 0)),
            pl.BlockSpec((NUM_EMB, DIM), lambda i: (0, 0)),
        ],
        out_specs=[
            pl.BlockSpec((TILE_N, DIM), lambda i: (i, 0)),
            pl.BlockSpec((1, 1, TILE_N), lambda i: (i, 0, 0)),
            pl.BlockSpec(memory_space=pltpu.SMEM),
            pl.BlockSpec(memory_space=pltpu.SMEM),
        ],
        out_shape=[
            jax.ShapeDtypeStruct((n, DIM), jnp.float32),
            jax.ShapeDtypeStruct((n // TILE_N, 1, TILE_N), jnp.int32),
            jax.ShapeDtypeStruct((1, 1), jnp.float32),
            jax.ShapeDtypeStruct((1, 1), jnp.float32),
        ],
        scratch_shapes=[
            pltpu.VMEM((NUM_EMB, DIM + 1), jnp.float32),
            pltpu.VMEM((1, NUM_EMB), jnp.float32),
            pltpu.VMEM((1, NUM_EMB), jnp.float32),
            pltpu.SMEM((1,), jnp.float32),
        ],
        compiler_params=pltpu.CompilerParams(
            dimension_semantics=("arbitrary",)),
    )(flat, codebook)
    return (q.reshape(inputs.shape), loss[0, 0], perp[0, 0],
            idx3.reshape(-1))


# speedup vs baseline: 1.7623x; 1.0693x over previous
import jax
import jax.numpy as jnp
from jax import lax
from jax.experimental import pallas as pl
from jax.experimental.pallas import tpu as pltpu

NUM_EMB = 1024
DIM = 64
COMMIT = 0.25
TILE_N = 2304


def _vq_body(x_ref, cb_ref, q_ref, idx_ref, loss_ref, perp_ref,
             cb2h_ref, fiota_ref, counts_ref, lsum_ref):
    step = pl.program_id(0)
    nsteps = pl.num_programs(0)
    x = x_ref[...]
    cb = cb_ref[...]

    @pl.when(step == 0)
    def _():
        cb2h_ref[...] = 0.5 * jnp.sum(cb * cb, axis=1, keepdims=True)
        fiota_ref[...] = lax.broadcasted_iota(
            jnp.int32, (NUM_EMB, 1), 0).astype(jnp.float32)

    xcT = lax.dot_general(cb, x, (((1,), (1,)), ((), ())),
                          preferred_element_type=jnp.float32)
    sT = xcT - cb2h_ref[...]
    smax = jnp.max(sT, axis=0, keepdims=True)
    fiota = fiota_ref[...]
    idx_row = jnp.min(jnp.where(sT == smax, fiota, 131072.0),
                      axis=0, keepdims=True)
    idx_ref[0, :, :] = idx_row.astype(jnp.int32)
    onehotT = (fiota == idx_row).astype(jnp.float32)
    q_ref[...] = lax.dot_general(onehotT, cb, (((0,), (0,)), ((), ())),
                                 preferred_element_type=jnp.float32)
    part_loss = jnp.sum(x * x) - 2.0 * jnp.sum(smax)
    ones_col = jnp.ones((TILE_N, 1), jnp.float32)
    part_counts = lax.dot_general(onehotT, ones_col, (((1,), (0,)), ((), ())),
                                  preferred_element_type=jnp.float32)

    @pl.when(step == 0)
    def _():
        counts_ref[...] = part_counts
        lsum_ref[0] = part_loss

    @pl.when(step != 0)
    def _():
        counts_ref[...] += part_counts
        lsum_ref[0] += part_loss

    @pl.when(step == nsteps - 1)
    def _():
        n_total = nsteps * TILE_N
        p = counts_ref[...] * (1.0 / n_total)
        perp_ref[0, 0] = jnp.exp(-jnp.sum(p * jnp.log(p + 1e-10)))
        loss_ref[0, 0] = (1.0 + COMMIT) * lsum_ref[0] / (n_total * DIM)


def kernel(inputs, codebook):
    flat = inputs.reshape(-1, DIM)
    n = flat.shape[0]
    grid = (n // TILE_N,)
    q, idx3, loss, perp = pl.pallas_call(
        _vq_body,
        grid=grid,
        in_specs=[
            pl.BlockSpec((TILE_N, DIM), lambda i: (i, 0)),
            pl.BlockSpec((NUM_EMB, DIM), lambda i: (0, 0)),
        ],
        out_specs=[
            pl.BlockSpec((TILE_N, DIM), lambda i: (i, 0)),
            pl.BlockSpec((1, 1, TILE_N), lambda i: (i, 0, 0)),
            pl.BlockSpec(memory_space=pltpu.SMEM),
            pl.BlockSpec(memory_space=pltpu.SMEM),
        ],
        out_shape=[
            jax.ShapeDtypeStruct((n, DIM), jnp.float32),
            jax.ShapeDtypeStruct((n // TILE_N, 1, TILE_N), jnp.int32),
            jax.ShapeDtypeStruct((1, 1), jnp.float32),
            jax.ShapeDtypeStruct((1, 1), jnp.float32),
        ],
        scratch_shapes=[
            pltpu.VMEM((NUM_EMB, 1), jnp.float32),
            pltpu.VMEM((NUM_EMB, 1), jnp.float32),
            pltpu.VMEM((NUM_EMB, 1), jnp.float32),
            pltpu.SMEM((1,), jnp.float32),
        ],
        compiler_params=pltpu.CompilerParams(
            dimension_semantics=("arbitrary",)),
    )(flat, codebook)
    return (q.reshape(inputs.shape), loss[0, 0], perp[0, 0],
            idx3.reshape(-1))
